# baseline (speedup 1.0000x reference)
import functools

import jax
import jax.numpy as jnp
from jax import lax
from jax.experimental import pallas as pl
from jax.experimental.pallas import tpu as pltpu
from jax.experimental.pallas import tpu_sc as plsc

N_TOK = 4096
DIM = 32
K_CODES = 1024
BN = 512
GRID = N_TOK // BN

_NC = 1
_NS = 16
_NW = _NC * _NS
_BPW = N_TOK // _NW
_L = 16

_LN2 = 0.6931471805599453


def _vq_body(x_ref, cb_ref, idx_ref, loss_ref, caug_ref, lsum_ref):
    i = pl.program_id(0)
    x = x_ref[...]

    @pl.when(i == 0)
    def _init():
        cb = cb_ref[...]
        caug_ref[:, :DIM] = -2.0 * cb
        caug_ref[:, DIM:] = jnp.sum(cb * cb, axis=1, keepdims=True)
        lsum_ref[...] = jnp.zeros_like(lsum_ref)

    xaug = jnp.concatenate(
        [x, jnp.ones((BN, 1), jnp.float32)], axis=1)
    d = jax.lax.dot_general(
        xaug, caug_ref[...], dimension_numbers=(((1,), (1,)), ((), ())),
        preferred_element_type=jnp.float32,
        precision=jax.lax.Precision.HIGHEST)

    mind = jnp.min(d, axis=1, keepdims=True)
    kiota = jax.lax.broadcasted_iota(jnp.int32, d.shape, 1)
    idx = jnp.min(jnp.where(d == mind, kiota, K_CODES), axis=1,
                  keepdims=True)
    idx_ref[...] = idx

    lsum_ref[...] += (jnp.sum(x * x, axis=(0, 1), keepdims=True)
                      + jnp.sum(mind, axis=(0, 1), keepdims=True))

    @pl.when(i == GRID - 1)
    def _finalize():
        loss_ref[...] = lsum_ref[...] / (N_TOK * DIM)


def _vq_tc(input_data, codebooks):
    return pl.pallas_call(
        _vq_body,
        grid=(GRID,),
        in_specs=[
            pl.BlockSpec((BN, DIM), lambda i: (i, 0)),
            pl.BlockSpec((K_CODES, DIM), lambda i: (0, 0)),
        ],
        out_specs=[
            pl.BlockSpec((BN, 1), lambda i: (i, 0)),
            pl.BlockSpec((1, 1), lambda i: (0, 0)),
        ],
        out_shape=[
            jax.ShapeDtypeStruct((N_TOK, 1), jnp.int32),
            jax.ShapeDtypeStruct((1, 1), jnp.float32),
        ],
        scratch_shapes=[
            pltpu.VMEM((K_CODES, DIM + 1), jnp.float32),
            pltpu.VMEM((1, 1), jnp.float32),
        ],
    )(input_data, codebooks)


def _ln(y):
    bits = plsc.bitcast(y, jnp.int32)
    e = ((bits >> 23) & 0xFF) - 127
    m = plsc.bitcast((bits & 0x007FFFFF) | 0x3F800000, jnp.float32)
    s = (m - 1.0) / (m + 1.0)
    s2 = s * s
    lnm = 2.0 * s * (1.0 + s2 * (1.0 / 3.0 + s2 * (0.2 + s2 * (1.0 / 7.0))))
    return e.astype(jnp.float32) * _LN2 + lnm


@functools.partial(
    pl.kernel,
    mesh=plsc.VectorSubcoreMesh(core_axis_name="c", subcore_axis_name="s", num_cores=1),
    out_type=[
        jax.ShapeDtypeStruct((N_TOK, DIM), jnp.float32),
        jax.ShapeDtypeStruct((_L,), jnp.float32),
    ],
    scratch_types=[
        pltpu.VMEM((_BPW,), jnp.int32),
        pltpu.VMEM((_BPW, DIM), jnp.float32),
        pltpu.VMEM((N_TOK,), jnp.int32),
        pltpu.VMEM((K_CODES,), jnp.float32),
        pltpu.VMEM((_L,), jnp.float32),
        pltpu.SemaphoreType.DMA,
    ],
    compiler_params=pltpu.CompilerParams(use_tc_tiling_on_sc=False,
                                         needs_layout_passes=False),
)
def _sc_gather_stats(cb_hbm, idx_hbm, out_hbm, perp_hbm,
                     idx_v, rows_v, allidx_v, cnt_v, tmp_v, sem):
    wid = lax.axis_index("s") * _NC + lax.axis_index("c")
    base = wid * _BPW

    pltpu.sync_copy(idx_hbm.at[pl.ds(base, _BPW)], idx_v)
    gather = pltpu.async_copy(cb_hbm.at[idx_v], rows_v, sem)

    @pl.when(wid == 0)
    def _stats():
        pltpu.sync_copy(idx_hbm, allidx_v)
        zeros = jnp.zeros((_L,), jnp.float32)
        for g in range(K_CODES // _L):
            cnt_v[pl.ds(g * _L, _L)] = zeros

        ones = jnp.ones((_L,), jnp.float32)

        def _hist(g, _):
            for u in range(8):
                iv = allidx_v[pl.ds((g * 8 + u) * _L, _L)]
                plsc.addupdate_scatter(cnt_v, [iv], ones)
            return 0

        lax.fori_loop(0, N_TOK // (_L * 8), _hist, 0)

        def _ent(g, acc):
            p = cnt_v[pl.ds(g * _L, _L)] * (1.0 / N_TOK)
            return acc + p * _ln(p + 1e-10)

        acc = lax.fori_loop(0, K_CODES // _L, _ent,
                            jnp.zeros((_L,), jnp.float32))
        ent = -jnp.sum(acc)
        tmp_v[...] = jnp.exp(jnp.full((_L,), ent, jnp.float32))
        pltpu.sync_copy(tmp_v, perp_hbm)

    gather.wait()
    pltpu.sync_copy(rows_v, out_hbm.at[pl.ds(base, _BPW)])


def kernel(input_data, codebooks):
    idx, loss = _vq_tc(input_data, codebooks)
    idx_flat = jnp.reshape(idx, (N_TOK,))
    q, perp = _sc_gather_stats(codebooks, idx_flat)
    return (q, jnp.reshape(loss, ()), jnp.reshape(perp[:1], ()), idx_flat)


# speedup vs baseline: 1.0168x; 1.0168x over previous
import functools

import jax
import jax.numpy as jnp
from jax import lax
from jax.experimental import pallas as pl
from jax.experimental.pallas import tpu as pltpu
from jax.experimental.pallas import tpu_sc as plsc

N_TOK = 4096
DIM = 32
K_CODES = 1024
BN = 1024
GRID = N_TOK // BN

_NC = 1
_NS = 8
_NW = _NC * _NS
_BPW = N_TOK // _NW
_L = 16

_LN2 = 0.6931471805599453


def _vq_body(x_ref, cb_ref, idx_ref, loss_ref, caug_ref, lsum_ref):
    i = pl.program_id(0)
    x = x_ref[...]

    @pl.when(i == 0)
    def _init():
        cb = cb_ref[...]
        caug_ref[:, :DIM] = -2.0 * cb
        caug_ref[:, DIM:] = jnp.sum(cb * cb, axis=1, keepdims=True)
        lsum_ref[...] = jnp.zeros_like(lsum_ref)

    xaug = jnp.concatenate(
        [x, jnp.ones((BN, 1), jnp.float32)], axis=1)
    d = jax.lax.dot_general(
        xaug, caug_ref[...], dimension_numbers=(((1,), (1,)), ((), ())),
        preferred_element_type=jnp.float32,
        precision=jax.lax.Precision.HIGHEST)

    mind = jnp.min(d, axis=1, keepdims=True)
    kiota = jax.lax.broadcasted_iota(jnp.int32, d.shape, 1)
    idx = jnp.min(jnp.where(d == mind, kiota, K_CODES), axis=1,
                  keepdims=True)
    idx_ref[...] = idx

    lsum_ref[...] += (jnp.sum(x * x, axis=(0, 1), keepdims=True)
                      + jnp.sum(mind, axis=(0, 1), keepdims=True))

    @pl.when(i == GRID - 1)
    def _finalize():
        loss_ref[...] = lsum_ref[...] / (N_TOK * DIM)


def _vq_tc(input_data, codebooks):
    return pl.pallas_call(
        _vq_body,
        grid=(GRID,),
        in_specs=[
            pl.BlockSpec((BN, DIM), lambda i: (i, 0)),
            pl.BlockSpec((K_CODES, DIM), lambda i: (0, 0)),
        ],
        out_specs=[
            pl.BlockSpec((BN, 1), lambda i: (i, 0)),
            pl.BlockSpec((1, 1), lambda i: (0, 0)),
        ],
        out_shape=[
            jax.ShapeDtypeStruct((N_TOK, 1), jnp.int32),
            jax.ShapeDtypeStruct((1, 1), jnp.float32),
        ],
        scratch_shapes=[
            pltpu.VMEM((K_CODES, DIM + 1), jnp.float32),
            pltpu.VMEM((1, 1), jnp.float32),
        ],
    )(input_data, codebooks)


def _ln(y):
    bits = plsc.bitcast(y, jnp.int32)
    e = ((bits >> 23) & 0xFF) - 127
    m = plsc.bitcast((bits & 0x007FFFFF) | 0x3F800000, jnp.float32)
    s = (m - 1.0) / (m + 1.0)
    s2 = s * s
    lnm = 2.0 * s * (1.0 + s2 * (1.0 / 3.0 + s2 * (0.2 + s2 * (1.0 / 7.0))))
    return e.astype(jnp.float32) * _LN2 + lnm


@functools.partial(
    pl.kernel,
    mesh=plsc.VectorSubcoreMesh(core_axis_name="c", subcore_axis_name="s", num_cores=1, num_subcores=8),
    out_type=[
        jax.ShapeDtypeStruct((N_TOK, DIM), jnp.float32),
        jax.ShapeDtypeStruct((_L,), jnp.float32),
    ],
    scratch_types=[
        pltpu.VMEM((_BPW,), jnp.int32),
        pltpu.VMEM((_BPW, DIM), jnp.float32),
        pltpu.VMEM((N_TOK,), jnp.int32),
        pltpu.VMEM((K_CODES,), jnp.float32),
        pltpu.VMEM((_L,), jnp.float32),
        pltpu.SemaphoreType.DMA,
    ],
    compiler_params=pltpu.CompilerParams(use_tc_tiling_on_sc=False,
                                         needs_layout_passes=False),
)
def _sc_gather_stats(cb_hbm, idx_hbm, out_hbm, perp_hbm,
                     idx_v, rows_v, allidx_v, cnt_v, tmp_v, sem):
    wid = lax.axis_index("s") * _NC + lax.axis_index("c")
    base = wid * _BPW

    pltpu.sync_copy(idx_hbm.at[pl.ds(base, _BPW)], idx_v)
    gather = pltpu.async_copy(cb_hbm.at[idx_v], rows_v, sem)

    @pl.when(wid == 0)
    def _stats():
        pltpu.sync_copy(idx_hbm, allidx_v)
        zeros = jnp.zeros((_L,), jnp.float32)
        for g in range(K_CODES // _L):
            cnt_v[pl.ds(g * _L, _L)] = zeros

        ones = jnp.ones((_L,), jnp.float32)

        def _hist(g, _):
            for u in range(8):
                iv = allidx_v[pl.ds((g * 8 + u) * _L, _L)]
                plsc.addupdate_scatter(cnt_v, [iv], ones)
            return 0

        lax.fori_loop(0, N_TOK // (_L * 8), _hist, 0)

        def _ent(g, acc):
            p = cnt_v[pl.ds(g * _L, _L)] * (1.0 / N_TOK)
            return acc + p * _ln(p + 1e-10)

        acc = lax.fori_loop(0, K_CODES // _L, _ent,
                            jnp.zeros((_L,), jnp.float32))
        ent = -jnp.sum(acc)
        tmp_v[...] = jnp.exp(jnp.full((_L,), ent, jnp.float32))
        pltpu.sync_copy(tmp_v, perp_hbm)

    gather.wait()
    pltpu.sync_copy(rows_v, out_hbm.at[pl.ds(base, _BPW)])


def kernel(input_data, codebooks):
    idx, loss = _vq_tc(input_data, codebooks)
    idx_flat = jnp.reshape(idx, (N_TOK,))
    q, perp = _sc_gather_stats(codebooks, idx_flat)
    return (q, jnp.reshape(loss, ()), jnp.reshape(perp[:1], ()), idx_flat)


# speedup vs baseline: 1.0246x; 1.0077x over previous
import functools

import jax
import jax.numpy as jnp
from jax import lax
from jax.experimental import pallas as pl
from jax.experimental.pallas import tpu as pltpu
from jax.experimental.pallas import tpu_sc as plsc

N_TOK = 4096
DIM = 32
K_CODES = 1024
BN = 1024
GRID = N_TOK // BN

_NC = 1
_NS = 16
_NW = _NC * _NS
_BPW = N_TOK // _NW
_L = 16

_LN2 = 0.6931471805599453


def _vq_body(x_ref, cb_ref, idx_ref, loss_ref, caug_ref, lsum_ref):
    i = pl.program_id(0)
    x = x_ref[...]

    @pl.when(i == 0)
    def _init():
        cb = cb_ref[...]
        caug_ref[:, :DIM] = -2.0 * cb
        caug_ref[:, DIM:] = jnp.sum(cb * cb, axis=1, keepdims=True)
        lsum_ref[...] = jnp.zeros_like(lsum_ref)

    xaug = jnp.concatenate(
        [x, jnp.ones((BN, 1), jnp.float32)], axis=1)
    d = jax.lax.dot_general(
        xaug, caug_ref[...], dimension_numbers=(((1,), (1,)), ((), ())),
        preferred_element_type=jnp.float32,
        precision=jax.lax.Precision.HIGHEST)

    mind = jnp.min(d, axis=1, keepdims=True)
    kiota = jax.lax.broadcasted_iota(jnp.int32, d.shape, 1)
    idx = jnp.min(jnp.where(d == mind, kiota, K_CODES), axis=1,
                  keepdims=True)
    idx_ref[...] = idx

    lsum_ref[...] += (jnp.sum(x * x, axis=(0, 1), keepdims=True)
                      + jnp.sum(mind, axis=(0, 1), keepdims=True))

    @pl.when(i == GRID - 1)
    def _finalize():
        loss_ref[...] = lsum_ref[...] / (N_TOK * DIM)


def _vq_tc(input_data, codebooks):
    return pl.pallas_call(
        _vq_body,
        grid=(GRID,),
        in_specs=[
            pl.BlockSpec((BN, DIM), lambda i: (i, 0)),
            pl.BlockSpec((K_CODES, DIM), lambda i: (0, 0)),
        ],
        out_specs=[
            pl.BlockSpec((BN, 1), lambda i: (i, 0)),
            pl.BlockSpec((1, 1), lambda i: (0, 0)),
        ],
        out_shape=[
            jax.ShapeDtypeStruct((N_TOK, 1), jnp.int32),
            jax.ShapeDtypeStruct((1, 1), jnp.float32),
        ],
        scratch_shapes=[
            pltpu.VMEM((K_CODES, DIM + 1), jnp.float32),
            pltpu.VMEM((1, 1), jnp.float32),
        ],
    )(input_data, codebooks)


def _ln(y):
    bits = plsc.bitcast(y, jnp.int32)
    e = ((bits >> 23) & 0xFF) - 127
    m = plsc.bitcast((bits & 0x007FFFFF) | 0x3F800000, jnp.float32)
    s = (m - 1.0) / (m + 1.0)
    s2 = s * s
    lnm = 2.0 * s * (1.0 + s2 * (1.0 / 3.0 + s2 * (0.2 + s2 * (1.0 / 7.0))))
    return e.astype(jnp.float32) * _LN2 + lnm


@functools.partial(
    pl.kernel,
    mesh=plsc.VectorSubcoreMesh(core_axis_name="c", subcore_axis_name="s", num_cores=1),
    out_type=[
        jax.ShapeDtypeStruct((N_TOK, DIM), jnp.float32),
        jax.ShapeDtypeStruct((_L,), jnp.float32),
    ],
    scratch_types=[
        pltpu.VMEM((_BPW,), jnp.int32),
        pltpu.VMEM((_BPW, DIM), jnp.float32),
        pltpu.VMEM((N_TOK,), jnp.int32),
        pltpu.VMEM((K_CODES,), jnp.float32),
        pltpu.VMEM((_L,), jnp.float32),
        pltpu.SemaphoreType.DMA,
    ],
    compiler_params=pltpu.CompilerParams(use_tc_tiling_on_sc=False,
                                         needs_layout_passes=False),
)
def _sc_gather_stats(cb_hbm, idx_hbm, out_hbm, perp_hbm,
                     idx_v, rows_v, allidx_v, cnt_v, tmp_v, sem):
    wid = lax.axis_index("s") * _NC + lax.axis_index("c")
    base = wid * _BPW

    pltpu.sync_copy(idx_hbm.at[pl.ds(base, _BPW)], idx_v)
    gather = pltpu.async_copy(cb_hbm.at[idx_v], rows_v, sem)

    @pl.when(wid == 0)
    def _stats():
        pltpu.sync_copy(idx_hbm, allidx_v)
        zeros = jnp.zeros((_L,), jnp.float32)
        for g in range(K_CODES // _L):
            cnt_v[pl.ds(g * _L, _L)] = zeros

        ones = jnp.ones((_L,), jnp.float32)

        def _hist(g, _):
            for u in range(8):
                iv = allidx_v[pl.ds((g * 8 + u) * _L, _L)]
                plsc.addupdate_scatter(cnt_v, [iv], ones)
            return 0

        lax.fori_loop(0, N_TOK // (_L * 8), _hist, 0)

        def _ent(g, acc):
            p = cnt_v[pl.ds(g * _L, _L)] * (1.0 / N_TOK)
            return acc + p * _ln(p + 1e-10)

        acc = lax.fori_loop(0, K_CODES // _L, _ent,
                            jnp.zeros((_L,), jnp.float32))
        ent = -jnp.sum(acc)
        tmp_v[...] = jnp.exp(jnp.full((_L,), ent, jnp.float32))
        pltpu.sync_copy(tmp_v, perp_hbm)

    gather.wait()
    pltpu.sync_copy(rows_v, out_hbm.at[pl.ds(base, _BPW)])


def kernel(input_data, codebooks):
    idx, loss = _vq_tc(input_data, codebooks)
    idx_flat = jnp.reshape(idx, (N_TOK,))
    q, perp = _sc_gather_stats(codebooks, idx_flat)
    return (q, jnp.reshape(loss, ()), jnp.reshape(perp[:1], ()), idx_flat)
